# baseline (speedup 1.0000x reference)
import jax
import jax.numpy as jnp
from jax import lax
from jax.experimental import pallas as pl
from jax.experimental.pallas import tpu as pltpu
from jax.experimental.pallas import tpu_sc as plsc

B, N = 4, 2048
D2D, D3D, DF, K, GH = 128, 128, 256, 8, 128
DH = 4 * DF
GATE_TEMP = 1.2
TOK = B * N
TBLK = 512
NC, NS, L = 2, 16, 16
TPW = TOK // (NC * NS)
HTOK = TOK // NC
EBLK = 512
NBH = 32
NB = 2 * NBH
HALF_ROWS = NBH * EBLK
DISP_ROWS = NB * EBLK
LOG2E = 9


def _ln(x, g, b):
    m = x.mean(axis=-1, keepdims=True)
    v = ((x - m) ** 2).mean(axis=-1, keepdims=True)
    return (x - m) * jax.lax.rsqrt(v + 1e-5) * g + b


def _gelu(x):
    return x * 0.5 * (1.0 + jax.lax.erf(x * 0.7071067811865476))


def _bcast_lane(v, lane):
    idx = jnp.full((L, 1), lane, jnp.int32)
    return lax.gather(
        v, idx,
        lax.GatherDimensionNumbers(offset_dims=(), collapsed_slice_dims=(0,),
                                   start_index_map=(0,)),
        (1,), mode=lax.GatherScatterMode.PROMISE_IN_BOUNDS)


def _prelude_body(h2d_ref, h3d_ref, W_in_ref, b_in_ref, g_ln_g_ref, g_ln_b_ref,
                  g_W1_ref, g_b1_ref, g_W2_ref, g_b2_ref, s_ln_g_ref,
                  s_ln_b_ref, s_W_ref, s_b_ref,
                  base_ref, e1_ref, e2_ref, w1_ref, w2_ref):
    f32 = jnp.float32
    x = (jnp.dot(h2d_ref[...], W_in_ref[:D2D, :], preferred_element_type=f32)
         + jnp.dot(h3d_ref[...], W_in_ref[D2D:, :], preferred_element_type=f32)
         + b_in_ref[...])
    base = jnp.dot(_gelu(_ln(x, s_ln_g_ref[...], s_ln_b_ref[...])), s_W_ref[...],
                   preferred_element_type=f32) + s_b_ref[...]
    base_ref[...] = base
    g = _ln(base, g_ln_g_ref[...], g_ln_b_ref[...])
    g = _gelu(jnp.dot(g, g_W1_ref[...], preferred_element_type=f32) + g_b1_ref[...])
    logits = jnp.dot(g, g_W2_ref[...], preferred_element_type=f32) + g_b2_ref[...]
    scores = logits / GATE_TEMP
    kio = jax.lax.broadcasted_iota(jnp.int32, scores.shape, 1)
    v1 = jnp.max(scores, axis=-1, keepdims=True)
    i1 = jnp.min(jnp.where(scores == v1, kio, K), axis=-1, keepdims=True)
    masked = jnp.where(kio == i1, -jnp.inf, scores)
    v2 = jnp.max(masked, axis=-1, keepdims=True)
    i2 = jnp.min(jnp.where(masked == v2, kio, K), axis=-1, keepdims=True)
    ex = jnp.exp(v2 - v1)
    p1 = 1.0 / (1.0 + ex)
    e1_ref[...] = i1
    e2_ref[...] = i2
    w1_ref[...] = p1
    w2_ref[...] = 1.0 - p1


def _route_body(e1_hbm, e2_hbm, base_hbm,
                disp_hbm, pos1_hbm, pos2_hbm, be_hbm, act_hbm, dsp_hbm,
                e1v, e2v, cntv, allc, basevm, p1a, p1b, p2a, p2b,
                bev, actv, dspv, rowsa, rowsb, shcnt, sem):
    c = lax.axis_index("c")
    s = lax.axis_index("s")
    tok0 = c * HTOK + s * TPW
    pltpu.sync_copy(e1_hbm.at[pl.ds(tok0, TPW)], e1v)
    pltpu.sync_copy(e2_hbm.at[pl.ds(tok0, TPW)], e2v)
    lane = lax.iota(jnp.int32, L)
    cnt = jnp.zeros((L,), jnp.int32)
    for g in range(TPW // L):
        for ev in (e1v, e2v):
            a = ev[pl.ds(g * L, L)]
            for e in range(K):
                pc = plsc.all_reduce_population_count(a == e)
                cnt = cnt + jnp.where(lane == e, pc, jnp.zeros((L,), jnp.int32))
    cntv[...] = cnt
    pltpu.sync_copy(cntv, shcnt.at[pl.ds(s * L, L)])
    plsc.subcore_barrier()
    pltpu.sync_copy(shcnt, allc)
    total = jnp.zeros((L,), jnp.int32)
    mypre = jnp.zeros((L,), jnp.int32)
    for t in range(NS):
        row = allc[pl.ds(t * L, L)]
        total = total + row
        mypre = mypre + jnp.where(t < s, row, jnp.zeros((L,), jnp.int32))
    padded = ((total + (EBLK - 1)) >> LOG2E) << LOG2E
    pcum = plsc.cumsum(padded)
    gstart = pcum - padded
    basevm[...] = c * HALF_ROWS + gstart + mypre

    @pl.when(s == 0)
    def _():
        nblk = padded >> LOG2E
        cumblk = plsc.cumsum(nblk)
        nact = _bcast_lane(cumblk, K - 1)
        for g2 in range(NBH // L):
            ivec = lax.iota(jnp.int32, L) + g2 * L
            be = jnp.zeros((L,), jnp.int32)
            for e in range(K):
                be = be + (ivec >= _bcast_lane(cumblk, e)).astype(jnp.int32)
            bev[pl.ds(g2 * L, L)] = jnp.minimum(be, K - 1)
            actv[pl.ds(g2 * L, L)] = (ivec < nact).astype(jnp.int32)
            dspv[pl.ds(g2 * L, L)] = jnp.minimum(ivec, nact - 1) + c * NBH
        pltpu.sync_copy(bev, be_hbm.at[pl.ds(c * NBH, NBH)])
        pltpu.sync_copy(actv, act_hbm.at[pl.ds(c * NBH, NBH)])
        pltpu.sync_copy(dspv, dsp_hbm.at[pl.ds(c * NBH, NBH)])

    for g in range(TPW // L):
        for ev, pda, pdb in ((e1v, p1a, p1b), (e2v, p2a, p2b)):
            a = ev[pl.ds(g * L, L)]
            bofs = plsc.load_gather(basevm, [a])
            r = jnp.zeros((L,), jnp.int32)
            upd = jnp.zeros((L,), jnp.int32)
            for e in range(K):
                m = a == e
                cm = plsc.cumsum(m.astype(jnp.int32))
                r = r + jnp.where(m, cm - 1, jnp.zeros((L,), jnp.int32))
                pc = plsc.all_reduce_population_count(m)
                upd = upd + jnp.where(lane == e, pc, jnp.zeros((L,), jnp.int32))
            pos = bofs + r
            if g < 8:
                pda[pl.ds(g * L, L)] = pos
            else:
                pdb[pl.ds((g - 8) * L, L)] = pos
            basevm[...] = basevm[...] + upd
    pltpu.sync_copy(p1a, pos1_hbm.at[pl.ds(tok0, 128)])
    pltpu.sync_copy(p1b, pos1_hbm.at[pl.ds(tok0 + 128, 128)])
    pltpu.sync_copy(p2a, pos2_hbm.at[pl.ds(tok0, 128)])
    pltpu.sync_copy(p2b, pos2_hbm.at[pl.ds(tok0 + 128, 128)])

    pltpu.sync_copy(base_hbm.at[pl.ds(tok0, 128)], rowsa)
    pltpu.sync_copy(base_hbm.at[pl.ds(tok0 + 128, 128)], rowsb)
    cps = [pltpu.async_copy(rowsa, disp_hbm.at[p1a], sem),
           pltpu.async_copy(rowsb, disp_hbm.at[p1b], sem),
           pltpu.async_copy(rowsa, disp_hbm.at[p2a], sem),
           pltpu.async_copy(rowsb, disp_hbm.at[p2b], sem)]
    for cp in cps:
        cp.wait()


def _expert_body(be_ref, act_ref, dsp_ref, disp_ref, elng_ref, elnb_ref,
                 W1_ref, b1_ref, W2_ref, b2_ref, y_ref):
    i = pl.program_id(0)

    @pl.when(act_ref[i] == 1)
    def _():
        f32 = jnp.float32
        h = _ln(disp_ref[...], elng_ref[0], elnb_ref[0])
        h = _gelu(jnp.dot(h, W1_ref[0], preferred_element_type=f32) + b1_ref[0])
        y_ref[...] = jnp.dot(h, W2_ref[0], preferred_element_type=f32) + b2_ref[0]


def _gather_body(y_hbm, pos1_hbm, pos2_hbm, Y1_hbm, Y2_hbm, pidx, yv, sem):
    c = lax.axis_index("c")
    s = lax.axis_index("s")
    tok0 = c * HTOK + s * TPW
    for pos_hbm, Y_hbm in ((pos1_hbm, Y1_hbm), (pos2_hbm, Y2_hbm)):
        for h in range(TPW // 128):
            o = tok0 + h * 128
            pltpu.sync_copy(pos_hbm.at[pl.ds(o, 128)], pidx)
            pltpu.async_copy(y_hbm.at[pidx], yv, sem).wait()
            pltpu.sync_copy(yv, Y_hbm.at[pl.ds(o, 128)])


def _final_body(Y1_ref, Y2_ref, w1_ref, w2_ref, base_ref, oW_ref, ob_ref,
                out_ref):
    comb = (w1_ref[...] * Y1_ref[...] + w2_ref[...] * Y2_ref[...]
            + base_ref[...])
    out_ref[...] = jnp.dot(comb, oW_ref[...],
                           preferred_element_type=jnp.float32) + ob_ref[...]


def kernel(h2d, h3d, W_in, b_in, g_ln_g, g_ln_b, g_W1, g_b1, g_W2, g_b2,
           s_ln_g, s_ln_b, s_W, s_b, e_ln_g, e_ln_b, e_W1, e_b1, e_W2, e_b2,
           o_W, o_b):
    f32, i32 = jnp.float32, jnp.int32
    h2 = h2d.reshape(TOK, D2D)
    h3 = h3d.reshape(TOK, D3D)
    nblk = TOK // TBLK

    def tok_spec(d):
        return pl.BlockSpec((TBLK, d), lambda i: (i, 0))

    def full_spec(arr):
        nd = arr.ndim
        return pl.BlockSpec(arr.shape, lambda i: (0,) * nd)

    pre_full = [W_in, b_in, g_ln_g, g_ln_b, g_W1, g_b1, g_W2, g_b2,
                s_ln_g, s_ln_b, s_W, s_b]
    base2d, e1c, e2c, w1c, w2c = pl.pallas_call(
        _prelude_body,
        grid=(nblk,),
        in_specs=[tok_spec(D2D), tok_spec(D3D)] + [full_spec(a) for a in pre_full],
        out_specs=[tok_spec(DF), tok_spec(1), tok_spec(1), tok_spec(1),
                   tok_spec(1)],
        out_shape=[jax.ShapeDtypeStruct((TOK, DF), f32),
                   jax.ShapeDtypeStruct((TOK, 1), i32),
                   jax.ShapeDtypeStruct((TOK, 1), i32),
                   jax.ShapeDtypeStruct((TOK, 1), f32),
                   jax.ShapeDtypeStruct((TOK, 1), f32)],
    )(h2, h3, *pre_full)
    e1 = e1c.reshape(TOK)
    e2 = e2c.reshape(TOK)

    route = pl.kernel(
        _route_body,
        out_type=[jax.ShapeDtypeStruct((DISP_ROWS, DF), f32),
                  jax.ShapeDtypeStruct((TOK,), i32),
                  jax.ShapeDtypeStruct((TOK,), i32),
                  jax.ShapeDtypeStruct((NB,), i32),
                  jax.ShapeDtypeStruct((NB,), i32),
                  jax.ShapeDtypeStruct((NB,), i32)],
        mesh=plsc.VectorSubcoreMesh(core_axis_name="c", subcore_axis_name="s", num_cores=2),
        compiler_params=pltpu.CompilerParams(needs_layout_passes=False),
        scratch_types=[pltpu.VMEM((TPW,), i32),
                       pltpu.VMEM((TPW,), i32),
                       pltpu.VMEM((L,), i32),
                       pltpu.VMEM((NS * L,), i32),
                       pltpu.VMEM((L,), i32),
                       pltpu.VMEM((128,), i32),
                       pltpu.VMEM((128,), i32),
                       pltpu.VMEM((128,), i32),
                       pltpu.VMEM((128,), i32),
                       pltpu.VMEM((NBH,), i32),
                       pltpu.VMEM((NBH,), i32),
                       pltpu.VMEM((NBH,), i32),
                       pltpu.VMEM((128, DF), f32),
                       pltpu.VMEM((128, DF), f32),
                       pltpu.VMEM_SHARED((NS * L,), i32),
                       pltpu.SemaphoreType.DMA],
    )
    disp, pos1, pos2, be, act, dsp = route(e1, e2, base2d)

    grid_spec = pltpu.PrefetchScalarGridSpec(
        num_scalar_prefetch=3,
        grid=(NB,),
        in_specs=[
            pl.BlockSpec((EBLK, DF), lambda i, be, act, dsp: (dsp[i], 0)),
            pl.BlockSpec((1, 1, DF), lambda i, be, act, dsp: (be[i], 0, 0)),
            pl.BlockSpec((1, 1, DF), lambda i, be, act, dsp: (be[i], 0, 0)),
            pl.BlockSpec((1, DF, DH), lambda i, be, act, dsp: (be[i], 0, 0)),
            pl.BlockSpec((1, 1, DH), lambda i, be, act, dsp: (be[i], 0, 0)),
            pl.BlockSpec((1, DH, DF), lambda i, be, act, dsp: (be[i], 0, 0)),
            pl.BlockSpec((1, 1, DF), lambda i, be, act, dsp: (be[i], 0, 0)),
        ],
        out_specs=pl.BlockSpec((EBLK, DF), lambda i, be, act, dsp: (dsp[i], 0)),
    )
    y = pl.pallas_call(
        _expert_body,
        grid_spec=grid_spec,
        out_shape=jax.ShapeDtypeStruct((DISP_ROWS, DF), f32),
    )(be, act, dsp, disp, e_ln_g.reshape(K, 1, DF), e_ln_b.reshape(K, 1, DF),
      e_W1, e_b1.reshape(K, 1, DH), e_W2, e_b2.reshape(K, 1, DF))

    comb = pl.kernel(
        _gather_body,
        out_type=[jax.ShapeDtypeStruct((TOK, DF), f32),
                  jax.ShapeDtypeStruct((TOK, DF), f32)],
        mesh=plsc.VectorSubcoreMesh(core_axis_name="c", subcore_axis_name="s", num_cores=2),
        compiler_params=pltpu.CompilerParams(needs_layout_passes=False),
        scratch_types=[pltpu.VMEM((128,), i32),
                       pltpu.VMEM((128, DF), f32),
                       pltpu.SemaphoreType.DMA],
    )
    Y1, Y2 = comb(y, pos1, pos2)

    out = pl.pallas_call(
        _final_body,
        grid=(nblk,),
        in_specs=[tok_spec(DF), tok_spec(DF), tok_spec(1), tok_spec(1),
                  tok_spec(DF), full_spec(o_W), full_spec(o_b)],
        out_specs=tok_spec(DF),
        out_shape=jax.ShapeDtypeStruct((TOK, DF), f32),
    )(Y1, Y2, w1c, w2c, base2d, o_W, o_b)
    return out.reshape(B, N, DF)
